# baseline (speedup 1.0000x reference)
import functools

import jax
import jax.numpy as jnp
from jax import lax
from jax.experimental import pallas as pl
from jax.experimental.pallas import tpu as pltpu
from jax.experimental.pallas import tpu_sc as plsc

NUM_CELLS = 500000
N_QUAD = 16
N_VERTICES = 250000
N_EDGES = 750000

TC_BLOCK = 2048
CELLS_PAD = 524288
TC_GRID = CELLS_PAD // TC_BLOCK
TC_LAST = (NUM_CELLS - 1) // TC_BLOCK

NC = 2
NS = 16
NW = NC * NS
CHUNK_W = 2048

SPAD = 9 * CELLS_PAD
WORDS_TILE = SPAD // NW
N_CHUNKS = WORDS_TILE // CHUNK_W

VACC = 250112
ACC = 1751040
ACC_TILE = ACC // NS
CP = 4560
N_CP = ACC_TILE // CP

CB_ROWS = 13680
CB_BLK = 1368


def _integral_body(fx_ref, det_ref, faces_ref, f2e_ref, o_ref,
                   wa_ref, wb_ref, wf_ref, sv_ref, si_ref, face_ref):
    i = pl.program_id(0)
    row = i * TC_BLOCK + lax.broadcasted_iota(jnp.int32, (TC_BLOCK, 1), 0)
    valid = row < NUM_CELLS
    fx = fx_ref[...]
    det = det_ref[...]
    of = o_ref[...].astype(jnp.float32)
    o9 = jnp.concatenate(
        [jnp.ones((TC_BLOCK, 3), jnp.float32), of, of], axis=1)
    ya = jnp.dot(fx, wa_ref[...], preferred_element_type=jnp.float32)
    yb = jnp.dot(fx, wb_ref[...], preferred_element_type=jnp.float32)
    yf = jnp.dot(fx, wf_ref[...], preferred_element_type=jnp.float32)
    sv_ref[...] = jnp.where(valid, (o9 * ya + (1.0 - o9) * yb) * det, 0.0)
    e2 = VACC + 2 * f2e_ref[...]
    si9 = jnp.concatenate([faces_ref[...], e2, e2 + 1], axis=1)
    si_ref[...] = jnp.where(valid, si9, 0)
    face_ref[...] = yf * det


def _integral(f_x, det2, faces, f2e, orient, wa, wb, wf):
    in_spec = lambda w: pl.BlockSpec(
        (TC_BLOCK, w), lambda i: (jnp.minimum(i, TC_LAST), 0))
    out_spec = lambda w: pl.BlockSpec((TC_BLOCK, w), lambda i: (i, 0))
    full_spec = lambda a: pl.BlockSpec(a.shape, lambda i: (0, 0))
    return pl.pallas_call(
        _integral_body,
        grid=(TC_GRID,),
        in_specs=[in_spec(N_QUAD), in_spec(1), in_spec(3), in_spec(3),
                  in_spec(3), full_spec(wa), full_spec(wb), full_spec(wf)],
        out_specs=[out_spec(9), out_spec(9), out_spec(1)],
        out_shape=[
            jax.ShapeDtypeStruct((CELLS_PAD, 9), jnp.float32),
            jax.ShapeDtypeStruct((CELLS_PAD, 9), jnp.int32),
            jax.ShapeDtypeStruct((CELLS_PAD, 1), jnp.float32),
        ],
    )(f_x, det2, faces, f2e, orient, wa, wb, wf)


def _scatter_body(svals, sidx, hout, acc,
                  ib0, vb0, ib1, vb1, cp, ls0, ls1, ssem, osem):
    c = lax.axis_index("c")
    s = lax.axis_index("s")
    wid = c * NS + s

    zvec = jnp.zeros((16,), jnp.float32)

    def zfill(i, carry):
        cp[pl.ds(i * 16, 16)] = zvec
        return carry
    lax.fori_loop(0, CP // 16, zfill, 0)
    for k in range(N_CP):
        pltpu.async_copy(cp, acc.at[pl.ds(s * ACC_TILE + k * CP, CP)], osem)
    for k in range(N_CP):
        pltpu.make_async_copy(
            cp, acc.at[pl.ds(s * ACC_TILE + k * CP, CP)], osem).wait()

    plsc.subcore_barrier()

    def load_start(t, ib, vb, sem):
        off = wid * WORDS_TILE + t * CHUNK_W
        pltpu.async_copy(sidx.at[pl.ds(off, CHUNK_W)], ib, sem)
        pltpu.async_copy(svals.at[pl.ds(off, CHUNK_W)], vb, sem)

    def load_wait(ib, vb, sem):
        pltpu.make_async_copy(sidx.at[pl.ds(0, CHUNK_W)], ib, sem).wait()
        pltpu.make_async_copy(svals.at[pl.ds(0, CHUNK_W)], vb, sem).wait()

    def scatter(ib, vb):
        pltpu.async_copy(vb, acc.at[ib], ssem, add=True)
        pltpu.make_async_copy(vb, acc.at[ib], ssem).wait()

    load_start(0, ib0, vb0, ls0)
    load_start(1, ib1, vb1, ls1)

    def pipe(p, carry):
        t0 = 2 * p
        load_wait(ib0, vb0, ls0)
        scatter(ib0, vb0)

        @pl.when(t0 + 2 < N_CHUNKS)
        def _():
            load_start(t0 + 2, ib0, vb0, ls0)
        load_wait(ib1, vb1, ls1)
        scatter(ib1, vb1)

        @pl.when(t0 + 3 < N_CHUNKS)
        def _():
            load_start(t0 + 3, ib1, vb1, ls1)
        return carry
    lax.fori_loop(0, N_CHUNKS // 2, pipe, 0)

    plsc.subcore_barrier()

    for k in range(N_CP):
        off = s * ACC_TILE + k * CP
        pltpu.async_copy(acc.at[pl.ds(off, CP)],
                         hout.at[pl.ds(c * ACC + off, CP)], osem)
    for k in range(N_CP):
        off = s * ACC_TILE + k * CP
        pltpu.make_async_copy(acc.at[pl.ds(off, CP)],
                              hout.at[pl.ds(c * ACC + off, CP)], osem).wait()


_scatter = functools.partial(
    pl.kernel,
    out_type=jax.ShapeDtypeStruct((NC * ACC,), jnp.float32),
    mesh=plsc.VectorSubcoreMesh(core_axis_name="c", subcore_axis_name="s"),
    compiler_params=pltpu.CompilerParams(use_tc_tiling_on_sc=False),
    scratch_types=[
        pltpu.VMEM_SHARED((ACC,), jnp.float32),
        pltpu.VMEM((CHUNK_W,), jnp.int32),
        pltpu.VMEM((CHUNK_W,), jnp.float32),
        pltpu.VMEM((CHUNK_W,), jnp.int32),
        pltpu.VMEM((CHUNK_W,), jnp.float32),
        pltpu.VMEM((CP,), jnp.float32),
        pltpu.SemaphoreType.DMA,
        pltpu.SemaphoreType.DMA,
        pltpu.SemaphoreType.DMA,
        pltpu.SemaphoreType.DMA,
    ],
)(_scatter_body)


def _combine_body(in_ref, out_ref):
    out_ref[...] = in_ref[0] + in_ref[1]


def _combine(hout2):
    return pl.pallas_call(
        _combine_body,
        grid=(CB_ROWS // CB_BLK,),
        in_specs=[pl.BlockSpec((2, CB_BLK, 128), lambda i: (0, i, 0))],
        out_specs=pl.BlockSpec((CB_BLK, 128), lambda i: (i, 0)),
        out_shape=jax.ShapeDtypeStruct((CB_ROWS, 128), jnp.float32),
    )(hout2)


def kernel(f_x, v_x, quad_weights, det_A, faces, faces_to_edges,
           faces_to_edge_orientation):
    w = v_x * quad_weights[None, :]
    wa = w[jnp.array([0, 1, 2, 3, 5, 7, 4, 6, 8])].T
    wb = w[jnp.array([0, 1, 2, 4, 6, 8, 3, 5, 7])].T
    wf = w[9:10].T
    det2 = det_A[:, None]

    sv, si, face_pad = _integral(
        f_x, det2, faces, faces_to_edges, faces_to_edge_orientation,
        wa, wb, wf)

    hout = _scatter(sv.reshape(SPAD), si.reshape(SPAD))
    fin = _combine(hout.reshape(NC, CB_ROWS, 128)).reshape(ACC)

    vertex_dofs = fin[:N_VERTICES]
    edge_dofs = fin[VACC:VACC + 2 * N_EDGES].reshape(N_EDGES, 2)
    return (vertex_dofs, edge_dofs, face_pad[:NUM_CELLS])


# speedup vs baseline: 1.4829x; 1.4829x over previous
import functools

import jax
import jax.numpy as jnp
from jax import lax
from jax.experimental import pallas as pl
from jax.experimental.pallas import tpu as pltpu
from jax.experimental.pallas import tpu_sc as plsc

NUM_CELLS = 500000
N_QUAD = 16
N_VERTICES = 250000
N_EDGES = 750000

TC_BLOCK = 4096
CELLS_PAD = 524288
TC_GRID = CELLS_PAD // TC_BLOCK
TC_LAST = (NUM_CELLS - 1) // TC_BLOCK

NC = 2
NS = 16
NW = NC * NS
CHUNK_W = 2048
PLANE_TILE = CELLS_PAD // NW
PLANE_CHUNKS = PLANE_TILE // CHUNK_W

VACC = 250112
ACC = 1751040
ACC_TILE = ACC // NS
CP = 4560
N_CP = ACC_TILE // CP


def _integral_body(fx_ref, det_ref, faces_ref, f2e_ref, o_ref,
                   wa_ref, wb_ref, wf_ref, *out_refs):
    i = pl.program_id(0)
    sv_refs = out_refs[0:9]
    si_refs = out_refs[9:18]
    face_ref = out_refs[18]
    cdim = (((1,), (1,)), ((), ()))
    fx = fx_ref[...]
    eye3 = jnp.eye(3, dtype=jnp.float32)
    ya = lax.dot_general(wa_ref[...], fx, cdim,
                         preferred_element_type=jnp.float32)
    yb = lax.dot_general(wb_ref[...], fx, cdim,
                         preferred_element_type=jnp.float32)
    yf = lax.dot_general(wf_ref[...], fx, cdim,
                         preferred_element_type=jnp.float32)
    hi = lax.Precision.HIGHEST
    ot = lax.dot_general(eye3, o_ref[...].astype(jnp.float32), cdim,
                         precision=hi,
                         preferred_element_type=jnp.float32)
    facest = lax.dot_general(eye3, faces_ref[...].astype(jnp.float32), cdim,
                             precision=hi,
                             preferred_element_type=jnp.float32)
    f2et = lax.dot_general(eye3, f2e_ref[...].astype(jnp.float32), cdim,
                           precision=hi,
                           preferred_element_type=jnp.float32)
    det = det_ref[...]
    o9 = jnp.concatenate(
        [jnp.ones((3, TC_BLOCK), jnp.float32), ot, ot], axis=0)
    sv = (o9 * ya + (1.0 - o9) * yb) * det
    e2 = float(VACC) + 2.0 * f2et
    si = jnp.concatenate([facest, e2, e2 + 1.0], axis=0)
    col = i * TC_BLOCK + lax.broadcasted_iota(jnp.int32, (1, TC_BLOCK), 1)
    valid = col < NUM_CELLS
    sv = jnp.where(valid, sv, 0.0)
    si9 = jnp.where(valid, si.astype(jnp.int32), 0)
    for k in range(9):
        sv_refs[k][...] = sv[k]
        si_refs[k][...] = si9[k]
    face_ref[...] = (yf * det)[0]


def _integral(f_x, det2, faces, f2e, orient, wa, wb, wf):
    row_spec = lambda w: pl.BlockSpec(
        (TC_BLOCK, w), lambda i: (jnp.minimum(i, TC_LAST), 0))
    lane_spec = pl.BlockSpec((1, TC_BLOCK),
                             lambda i: (0, jnp.minimum(i, TC_LAST)))
    out_spec = pl.BlockSpec((TC_BLOCK,), lambda i: (i,))
    full_spec = lambda a: pl.BlockSpec(a.shape, lambda i: (0, 0))
    return pl.pallas_call(
        _integral_body,
        grid=(TC_GRID,),
        in_specs=[row_spec(N_QUAD), lane_spec, row_spec(3), row_spec(3),
                  row_spec(3), full_spec(wa), full_spec(wb), full_spec(wf)],
        out_specs=[out_spec] * 19,
        out_shape=(
            [jax.ShapeDtypeStruct((CELLS_PAD,), jnp.float32)] * 9
            + [jax.ShapeDtypeStruct((CELLS_PAD,), jnp.int32)] * 9
            + [jax.ShapeDtypeStruct((CELLS_PAD,), jnp.float32)]
        ),
    )(f_x, det2, faces, f2e, orient, wa, wb, wf)


def _scatter_body(*refs):
    planes = [(refs[9 + k], refs[k]) for k in range(9)]
    hout = refs[18]
    acc, ib0, vb0, ib1, vb1, cp, ls0, ls1, ssem, osem = refs[19:]
    c = lax.axis_index("c")
    s = lax.axis_index("s")
    wid = c * NS + s

    zvec = jnp.zeros((16,), jnp.float32)

    def zfill(i, carry):
        cp[pl.ds(i * 16, 16)] = zvec
        return carry
    lax.fori_loop(0, CP // 16, zfill, 0)
    for k in range(N_CP):
        pltpu.async_copy(cp, acc.at[pl.ds(s * ACC_TILE + k * CP, CP)], osem)
    for k in range(N_CP):
        pltpu.make_async_copy(
            cp, acc.at[pl.ds(s * ACC_TILE + k * CP, CP)], osem).wait()

    plsc.subcore_barrier()

    base = wid * PLANE_TILE

    def load_start(iref, vref, t, ib, vb, sem):
        off = base + t * CHUNK_W
        pltpu.async_copy(iref.at[pl.ds(off, CHUNK_W)], ib, sem)
        pltpu.async_copy(vref.at[pl.ds(off, CHUNK_W)], vb, sem)

    def load_wait(iref, vref, ib, vb, sem):
        pltpu.make_async_copy(iref.at[pl.ds(0, CHUNK_W)], ib, sem).wait()
        pltpu.make_async_copy(vref.at[pl.ds(0, CHUNK_W)], vb, sem).wait()

    def scatter(ib, vb):
        pltpu.async_copy(vb, acc.at[ib], ssem, add=True)
        pltpu.make_async_copy(vb, acc.at[ib], ssem).wait()

    for iref, vref in planes:
        load_start(iref, vref, 0, ib0, vb0, ls0)
        load_start(iref, vref, 1, ib1, vb1, ls1)

        def pipe(p, carry, iref=iref, vref=vref):
            t0 = 2 * p
            load_wait(iref, vref, ib0, vb0, ls0)
            scatter(ib0, vb0)

            @pl.when(t0 + 2 < PLANE_CHUNKS)
            def _():
                load_start(iref, vref, t0 + 2, ib0, vb0, ls0)
            load_wait(iref, vref, ib1, vb1, ls1)
            scatter(ib1, vb1)

            @pl.when(t0 + 3 < PLANE_CHUNKS)
            def _():
                load_start(iref, vref, t0 + 3, ib1, vb1, ls1)
            return carry
        lax.fori_loop(0, PLANE_CHUNKS // 2, pipe, 0)

    plsc.subcore_barrier()

    for k in range(N_CP):
        off = s * ACC_TILE + k * CP
        pltpu.async_copy(acc.at[pl.ds(off, CP)],
                         hout.at[pl.ds(c * ACC + off, CP)], osem)
    for k in range(N_CP):
        off = s * ACC_TILE + k * CP
        pltpu.make_async_copy(acc.at[pl.ds(off, CP)],
                              hout.at[pl.ds(c * ACC + off, CP)], osem).wait()


_scatter = functools.partial(
    pl.kernel,
    out_type=jax.ShapeDtypeStruct((NC * ACC,), jnp.float32),
    mesh=plsc.VectorSubcoreMesh(core_axis_name="c", subcore_axis_name="s"),
    compiler_params=pltpu.CompilerParams(use_tc_tiling_on_sc=False),
    scratch_types=[
        pltpu.VMEM_SHARED((ACC,), jnp.float32),
        pltpu.VMEM((CHUNK_W,), jnp.int32),
        pltpu.VMEM((CHUNK_W,), jnp.float32),
        pltpu.VMEM((CHUNK_W,), jnp.int32),
        pltpu.VMEM((CHUNK_W,), jnp.float32),
        pltpu.VMEM((CP,), jnp.float32),
        pltpu.SemaphoreType.DMA,
        pltpu.SemaphoreType.DMA,
        pltpu.SemaphoreType.DMA,
        pltpu.SemaphoreType.DMA,
    ],
)(_scatter_body)


def _combine_body(h_ref, fp_ref, vert_ref, edge_ref, face_ref):
    vert_ref[...] = (h_ref[pl.ds(0, N_VERTICES)]
                     + h_ref[pl.ds(ACC, N_VERTICES)])
    edge_ref[...] = (h_ref[pl.ds(VACC, 2 * N_EDGES)]
                     + h_ref[pl.ds(ACC + VACC, 2 * N_EDGES)])
    face_ref[...] = fp_ref[pl.ds(0, NUM_CELLS)]


def _combine(hout, facep):
    whole = lambda n: pl.BlockSpec((n,), lambda: (0,))
    return pl.pallas_call(
        _combine_body,
        in_specs=[whole(NC * ACC), whole(CELLS_PAD)],
        out_specs=[whole(N_VERTICES), whole(2 * N_EDGES), whole(NUM_CELLS)],
        out_shape=[
            jax.ShapeDtypeStruct((N_VERTICES,), jnp.float32),
            jax.ShapeDtypeStruct((2 * N_EDGES,), jnp.float32),
            jax.ShapeDtypeStruct((NUM_CELLS,), jnp.float32),
        ],
    )(hout, facep)


def kernel(f_x, v_x, quad_weights, det_A, faces, faces_to_edges,
           faces_to_edge_orientation):
    w = v_x * quad_weights[None, :]
    wa = w[jnp.array([0, 1, 2, 3, 5, 7, 4, 6, 8])]
    wb = w[jnp.array([0, 1, 2, 4, 6, 8, 3, 5, 7])]
    wf = w[9:10]
    det2 = det_A[None, :]

    outs = _integral(f_x, det2, faces, faces_to_edges,
                     faces_to_edge_orientation, wa, wb, wf)

    hout = _scatter(*outs[0:18])
    vert, edge, face = _combine(hout, outs[18])
    return (vert, edge.reshape(N_EDGES, 2), face.reshape(NUM_CELLS, 1))


# speedup vs baseline: 1.4916x; 1.0059x over previous
import functools

import jax
import jax.numpy as jnp
from jax import lax
from jax.experimental import pallas as pl
from jax.experimental.pallas import tpu as pltpu
from jax.experimental.pallas import tpu_sc as plsc

NUM_CELLS = 500000
N_QUAD = 16
N_VERTICES = 250000
N_EDGES = 750000

TC_BLOCK = 8192
CELLS_PAD = 524288
TC_GRID = CELLS_PAD // TC_BLOCK
TC_LAST = (NUM_CELLS - 1) // TC_BLOCK

NC = 2
NS = 16
NW = NC * NS
CHUNK_W = 2048
PLANE_TILE = CELLS_PAD // NW
PLANE_CHUNKS = PLANE_TILE // CHUNK_W

VACC = 250112
ACC = 1751040
ACC_TILE = ACC // NS
CP = 4560
N_CP = ACC_TILE // CP


def _integral_body(fx_ref, det_ref, faces_ref, f2e_ref, o_ref,
                   wa_ref, wb_ref, wf_ref, *out_refs):
    i = pl.program_id(0)
    sv_refs = out_refs[0:9]
    si_refs = out_refs[9:18]
    face_ref = out_refs[18]
    cdim = (((1,), (1,)), ((), ()))
    fx = fx_ref[...]
    eye3 = jnp.eye(3, dtype=jnp.float32)
    ya = lax.dot_general(wa_ref[...], fx, cdim,
                         preferred_element_type=jnp.float32)
    yb = lax.dot_general(wb_ref[...], fx, cdim,
                         preferred_element_type=jnp.float32)
    yf = lax.dot_general(wf_ref[...], fx, cdim,
                         preferred_element_type=jnp.float32)
    hi = lax.Precision.HIGHEST
    ot = lax.dot_general(eye3, o_ref[...].astype(jnp.float32), cdim,
                         precision=hi,
                         preferred_element_type=jnp.float32)
    facest = lax.dot_general(eye3, faces_ref[...].astype(jnp.float32), cdim,
                             precision=hi,
                             preferred_element_type=jnp.float32)
    f2et = lax.dot_general(eye3, f2e_ref[...].astype(jnp.float32), cdim,
                           precision=hi,
                           preferred_element_type=jnp.float32)
    det = det_ref[...]
    o9 = jnp.concatenate(
        [jnp.ones((3, TC_BLOCK), jnp.float32), ot, ot], axis=0)
    sv = (o9 * ya + (1.0 - o9) * yb) * det
    e2 = float(VACC) + 2.0 * f2et
    si = jnp.concatenate([facest, e2, e2 + 1.0], axis=0)
    col = i * TC_BLOCK + lax.broadcasted_iota(jnp.int32, (1, TC_BLOCK), 1)
    valid = col < NUM_CELLS
    sv = jnp.where(valid, sv, 0.0)
    si9 = jnp.where(valid, si.astype(jnp.int32), 0)
    for k in range(9):
        sv_refs[k][...] = sv[k]
        si_refs[k][...] = si9[k]
    face_ref[...] = (yf * det)[0]


def _integral(f_x, det2, faces, f2e, orient, wa, wb, wf):
    row_spec = lambda w: pl.BlockSpec(
        (TC_BLOCK, w), lambda i: (jnp.minimum(i, TC_LAST), 0))
    lane_spec = pl.BlockSpec((1, TC_BLOCK),
                             lambda i: (0, jnp.minimum(i, TC_LAST)))
    out_spec = pl.BlockSpec((TC_BLOCK,), lambda i: (i,))
    full_spec = lambda a: pl.BlockSpec(a.shape, lambda i: (0, 0))
    return pl.pallas_call(
        _integral_body,
        grid=(TC_GRID,),
        in_specs=[row_spec(N_QUAD), lane_spec, row_spec(3), row_spec(3),
                  row_spec(3), full_spec(wa), full_spec(wb), full_spec(wf)],
        out_specs=[out_spec] * 19,
        out_shape=(
            [jax.ShapeDtypeStruct((CELLS_PAD,), jnp.float32)] * 9
            + [jax.ShapeDtypeStruct((CELLS_PAD,), jnp.int32)] * 9
            + [jax.ShapeDtypeStruct((CELLS_PAD,), jnp.float32)]
        ),
    )(f_x, det2, faces, f2e, orient, wa, wb, wf)


def _scatter_body(*refs):
    planes = [(refs[9 + k], refs[k]) for k in range(9)]
    hout = refs[18]
    acc, ib0, vb0, ib1, vb1, cp, ls0, ls1, ssem, osem = refs[19:]
    c = lax.axis_index("c")
    s = lax.axis_index("s")
    wid = c * NS + s

    zvec = jnp.zeros((16,), jnp.float32)

    def zfill(i, carry):
        cp[pl.ds(i * 16, 16)] = zvec
        return carry
    lax.fori_loop(0, CP // 16, zfill, 0)
    for k in range(N_CP):
        pltpu.async_copy(cp, acc.at[pl.ds(s * ACC_TILE + k * CP, CP)], osem)
    for k in range(N_CP):
        pltpu.make_async_copy(
            cp, acc.at[pl.ds(s * ACC_TILE + k * CP, CP)], osem).wait()

    plsc.subcore_barrier()

    base = wid * PLANE_TILE

    def load_start(iref, vref, t, ib, vb, sem):
        off = base + t * CHUNK_W
        pltpu.async_copy(iref.at[pl.ds(off, CHUNK_W)], ib, sem)
        pltpu.async_copy(vref.at[pl.ds(off, CHUNK_W)], vb, sem)

    def load_wait(iref, vref, ib, vb, sem):
        pltpu.make_async_copy(iref.at[pl.ds(0, CHUNK_W)], ib, sem).wait()
        pltpu.make_async_copy(vref.at[pl.ds(0, CHUNK_W)], vb, sem).wait()

    def scatter(ib, vb):
        pltpu.async_copy(vb, acc.at[ib], ssem, add=True)
        pltpu.make_async_copy(vb, acc.at[ib], ssem).wait()

    for iref, vref in planes:
        load_start(iref, vref, 0, ib0, vb0, ls0)
        load_start(iref, vref, 1, ib1, vb1, ls1)

        def pipe(p, carry, iref=iref, vref=vref):
            t0 = 2 * p
            load_wait(iref, vref, ib0, vb0, ls0)
            scatter(ib0, vb0)

            @pl.when(t0 + 2 < PLANE_CHUNKS)
            def _():
                load_start(iref, vref, t0 + 2, ib0, vb0, ls0)
            load_wait(iref, vref, ib1, vb1, ls1)
            scatter(ib1, vb1)

            @pl.when(t0 + 3 < PLANE_CHUNKS)
            def _():
                load_start(iref, vref, t0 + 3, ib1, vb1, ls1)
            return carry
        lax.fori_loop(0, PLANE_CHUNKS // 2, pipe, 0)

    plsc.subcore_barrier()

    for k in range(N_CP):
        off = s * ACC_TILE + k * CP
        pltpu.async_copy(acc.at[pl.ds(off, CP)],
                         hout.at[pl.ds(c * ACC + off, CP)], osem)
    for k in range(N_CP):
        off = s * ACC_TILE + k * CP
        pltpu.make_async_copy(acc.at[pl.ds(off, CP)],
                              hout.at[pl.ds(c * ACC + off, CP)], osem).wait()


_scatter = functools.partial(
    pl.kernel,
    out_type=jax.ShapeDtypeStruct((NC * ACC,), jnp.float32),
    mesh=plsc.VectorSubcoreMesh(core_axis_name="c", subcore_axis_name="s"),
    compiler_params=pltpu.CompilerParams(use_tc_tiling_on_sc=False),
    scratch_types=[
        pltpu.VMEM_SHARED((ACC,), jnp.float32),
        pltpu.VMEM((CHUNK_W,), jnp.int32),
        pltpu.VMEM((CHUNK_W,), jnp.float32),
        pltpu.VMEM((CHUNK_W,), jnp.int32),
        pltpu.VMEM((CHUNK_W,), jnp.float32),
        pltpu.VMEM((CP,), jnp.float32),
        pltpu.SemaphoreType.DMA,
        pltpu.SemaphoreType.DMA,
        pltpu.SemaphoreType.DMA,
        pltpu.SemaphoreType.DMA,
    ],
)(_scatter_body)


def _combine_body(h_ref, fp_ref, vert_ref, edge_ref, face_ref):
    vert_ref[...] = (h_ref[pl.ds(0, N_VERTICES)]
                     + h_ref[pl.ds(ACC, N_VERTICES)])
    edge_ref[...] = (h_ref[pl.ds(VACC, 2 * N_EDGES)]
                     + h_ref[pl.ds(ACC + VACC, 2 * N_EDGES)])
    face_ref[...] = fp_ref[pl.ds(0, NUM_CELLS)]


def _combine(hout, facep):
    whole = lambda n: pl.BlockSpec((n,), lambda: (0,))
    return pl.pallas_call(
        _combine_body,
        in_specs=[whole(NC * ACC), whole(CELLS_PAD)],
        out_specs=[whole(N_VERTICES), whole(2 * N_EDGES), whole(NUM_CELLS)],
        out_shape=[
            jax.ShapeDtypeStruct((N_VERTICES,), jnp.float32),
            jax.ShapeDtypeStruct((2 * N_EDGES,), jnp.float32),
            jax.ShapeDtypeStruct((NUM_CELLS,), jnp.float32),
        ],
    )(hout, facep)


def kernel(f_x, v_x, quad_weights, det_A, faces, faces_to_edges,
           faces_to_edge_orientation):
    w = v_x * quad_weights[None, :]
    wa = w[jnp.array([0, 1, 2, 3, 5, 7, 4, 6, 8])]
    wb = w[jnp.array([0, 1, 2, 4, 6, 8, 3, 5, 7])]
    wf = w[9:10]
    det2 = det_A[None, :]

    outs = _integral(f_x, det2, faces, faces_to_edges,
                     faces_to_edge_orientation, wa, wb, wf)

    hout = _scatter(*outs[0:18])
    vert, edge, face = _combine(hout, outs[18])
    return (vert, edge.reshape(N_EDGES, 2), face.reshape(NUM_CELLS, 1))
